# Initial kernel scaffold; baseline (speedup 1.0000x reference)
import jax
import jax.numpy as jnp
from jax.experimental import pallas as pl


def kernel(mask, table):
    raise NotImplementedError("write your pallas kernel here")


# speedup vs baseline: 8.1189x; 8.1189x over previous
import jax
import jax.numpy as jnp
from jax.experimental import pallas as pl


def _body(mask_ref, table_ref, out_ref):
    m = mask_ref[...].astype(jnp.float32)
    t = table_ref[...]
    t0 = t[0]
    diff = t[1] - t[0]
    out_ref[...] = t0[None, None, :] + m[:, :, None] * diff[None, None, :]


def kernel(mask, table):
    B, H = mask.shape
    D = table.shape[1]
    BLK = 128
    return pl.pallas_call(
        _body,
        grid=(B // BLK,),
        in_specs=[
            pl.BlockSpec((BLK, H), lambda i: (i, 0)),
            pl.BlockSpec((2, D), lambda i: (0, 0)),
        ],
        out_specs=pl.BlockSpec((BLK, H, D), lambda i: (i, 0, 0)),
        out_shape=jax.ShapeDtypeStruct((B, H, D), jnp.float32),
    )(mask, table)
